# baseline (speedup 1.0000x reference)
import jax
import jax.numpy as jnp
from jax.experimental import pallas as pl

_N = 1024
_C = 1000
_ROWS = 16384
_BR = 8


def _body(p_ref, l_ref, o_ref):
    i = pl.program_id(0)
    p = p_ref[...]
    l = l_ref[...]
    jidx = jax.lax.broadcasted_iota(jnp.int32, (_N, _N), 1)
    midx = jax.lax.broadcasted_iota(jnp.int32, (_N, _N), 0)
    tie = midx < jidx
    acc = jnp.float32(0.0)
    for rr in range(_BR):
        prow = p[rr:rr + 1, :]
        pcol = jnp.reshape(prow, (_N, 1))
        lrow = l[rr:rr + 1, :]
        lcol = jnp.reshape(lrow, (_N, 1))
        strict = pcol > prow
        eqtie = (pcol == prow) & tie
        G = (strict | eqtie).astype(jnp.float32)
        r = 1.0 + jnp.sum(G, axis=0, keepdims=True)
        c = 1.0 + jnp.sum(G * lcol, axis=0, keepdims=True)
        npos = jnp.sum(lrow)
        acc += jnp.sum(lrow * c / r) / npos
    contrib = jnp.reshape(acc * (1.0 / _ROWS), (1, 1))
    @pl.when(i == 0)
    def _():
        o_ref[...] = contrib
    @pl.when(i > 0)
    def _():
        o_ref[...] = o_ref[...] + contrib


def kernel(preds, labels):
    pp = jnp.pad(preds, ((0, 0), (0, _N - _C)), constant_values=-jnp.inf)
    lp = jnp.pad(labels, ((0, 0), (0, _N - _C)))
    out = pl.pallas_call(
        _body,
        grid=(_ROWS // _BR,),
        in_specs=[
            pl.BlockSpec((_BR, _N), lambda i: (i, 0)),
            pl.BlockSpec((_BR, _N), lambda i: (i, 0)),
        ],
        out_specs=pl.BlockSpec((1, 1), lambda i: (0, 0)),
        out_shape=jax.ShapeDtypeStruct((1, 1), jnp.float32),
    )(pp, lp)
    return out[0, 0]


# speedup vs baseline: 2.7141x; 2.7141x over previous
import jax
import jax.numpy as jnp
from jax.experimental import pallas as pl

_N = 1024
_C = 1000
_ROWS = 16384
_BR = 32


def _sort_stage(k, v, j, axis, shift, idx):
    low = (idx & j) == 0
    pk = jnp.where(low, jnp.roll(k, -shift, axis=axis),
                   jnp.roll(k, shift, axis=axis))
    pv = jnp.where(low, jnp.roll(v, -shift, axis=axis),
                   jnp.roll(v, shift, axis=axis))
    return pk, pv


def _body(p_ref, l_ref, o_ref):
    i = pl.program_id(0)
    k = p_ref[...]
    lab = l_ref[...]
    shape = k.shape
    sub = jax.lax.broadcasted_iota(jnp.int32, shape, 1)
    lane = jax.lax.broadcasted_iota(jnp.int32, shape, 2)
    idx = sub * 128 + lane
    v = idx * 2 + lab.astype(jnp.int32)

    for ksz_log in range(1, 11):
        ksz = 1 << ksz_log
        asc = (idx & ksz) == 0
        for j_log in range(ksz_log - 1, -1, -1):
            j = 1 << j_log
            if j < 128:
                axis, shift = 2, j
            else:
                axis, shift = 1, j // 128
            low = (idx & j) == 0
            pk = jnp.where(low, jnp.roll(k, -shift, axis=axis),
                           jnp.roll(k, shift, axis=axis))
            pv = jnp.where(low, jnp.roll(v, -shift, axis=axis),
                           jnp.roll(v, shift, axis=axis))
            after = (k < pk) | ((k == pk) & (v > pv))
            take_partner = after == (asc == low)
            k = jnp.where(take_partner, pk, k)
            v = jnp.where(take_partner, pv, v)

    ls = (v & 1).astype(jnp.float32)

    cum = ls
    for d in (1, 2, 4, 8, 16, 32, 64):
        cum = cum + jnp.where(lane >= d, jnp.roll(cum, d, axis=2), 0.0)
    tot = jax.lax.slice_in_dim(cum, 127, 128, axis=2)
    sub1 = jax.lax.broadcasted_iota(jnp.int32, tot.shape, 1)
    inc = tot
    for d in (1, 2, 4):
        inc = inc + jnp.where(sub1 >= d, jnp.roll(inc, d, axis=1), 0.0)
    cum = cum + (inc - tot)

    pos = (idx + 1).astype(jnp.float32)
    terms = ls * (cum / pos)
    row_sum = jnp.sum(terms, axis=(1, 2))
    npos = jnp.sum(ls, axis=(1, 2))
    acc = jnp.sum(row_sum / npos)

    contrib = jnp.reshape(acc * (1.0 / _ROWS), (1, 1))
    @pl.when(i == 0)
    def _():
        o_ref[...] = contrib
    @pl.when(i > 0)
    def _():
        o_ref[...] = o_ref[...] + contrib


def kernel(preds, labels):
    pp = jnp.pad(preds, ((0, 0), (0, _N - _C)), constant_values=-jnp.inf)
    lp = jnp.pad(labels, ((0, 0), (0, _N - _C)))
    pp = jnp.reshape(pp, (_ROWS, 8, 128))
    lp = jnp.reshape(lp, (_ROWS, 8, 128))
    out = pl.pallas_call(
        _body,
        grid=(_ROWS // _BR,),
        in_specs=[
            pl.BlockSpec((_BR, 8, 128), lambda i: (i, 0, 0)),
            pl.BlockSpec((_BR, 8, 128), lambda i: (i, 0, 0)),
        ],
        out_specs=pl.BlockSpec((1, 1), lambda i: (0, 0)),
        out_shape=jax.ShapeDtypeStruct((1, 1), jnp.float32),
    )(pp, lp)
    return out[0, 0]


# speedup vs baseline: 5.6852x; 2.0947x over previous
import jax
import jax.numpy as jnp
from jax.experimental import pallas as pl

_N = 1024
_C = 1000
_ROWS = 16384
_BR = 128


def _body(*refs):
    o_ref = refs[16]
    i = pl.program_id(0)
    lane = jax.lax.broadcasted_iota(jnp.int32, (_BR, 128), 1)

    pad7 = lane >= (_C - 7 * 128)
    K = [refs[g][...] for g in range(8)]
    K[7] = jnp.where(pad7, -jnp.inf, K[7])
    L = [refs[8 + g][...] for g in range(8)]
    L[7] = jnp.where(pad7, 0.0, L[7])
    V = [lane * 2 + (g * 256) + L[g].astype(jnp.int32)
         for g in range(8)]

    for ksz_log in range(1, 11):
        ksz = 1 << ksz_log
        for j_log in range(ksz_log - 1, -1, -1):
            j = 1 << j_log
            if j < 8:
                for g in range(8):
                    if g & j:
                        continue
                    h = g ^ j
                    a, b, va, vb = K[g], K[h], V[g], V[h]
                    after = (a < b) | ((a == b) & (va > vb))
                    if ksz < 8:
                        swap = (~after) if (g & ksz) else after
                    else:
                        asc = (lane & (ksz // 8)) == 0
                        swap = after == asc
                    K[g] = jnp.where(swap, b, a)
                    K[h] = jnp.where(swap, a, b)
                    V[g] = jnp.where(swap, vb, va)
                    V[h] = jnp.where(swap, va, vb)
            else:
                jl = j // 8
                asc = (lane & (ksz // 8)) == 0
                low = (lane & jl) == 0
                for g in range(8):
                    k, v = K[g], V[g]
                    pk = jnp.where(low, jnp.roll(k, -jl, axis=1),
                                   jnp.roll(k, jl, axis=1))
                    pv = jnp.where(low, jnp.roll(v, -jl, axis=1),
                                   jnp.roll(v, jl, axis=1))
                    after = (k < pk) | ((k == pk) & (v > pv))
                    take = after == (asc == low)
                    K[g] = jnp.where(take, pk, k)
                    V[g] = jnp.where(take, pv, v)

    ls = [(V[g] & 1).astype(jnp.float32) for g in range(8)]
    run = ls[0]
    cums = [run]
    for g in range(1, 8):
        run = run + ls[g]
        cums.append(run)
    inc = run
    for d in (1, 2, 4, 8, 16, 32, 64):
        inc = inc + jnp.where(lane >= d, jnp.roll(inc, d, axis=1), 0.0)
    excl = inc - run

    term_sum = jnp.zeros_like(run)
    npos = run * 0.0
    for g in range(8):
        pos = (lane * 8 + g + 1).astype(jnp.float32)
        cum = cums[g] + excl
        term_sum = term_sum + ls[g] * (cum / pos)
        npos = npos + ls[g]
    row_sum = jnp.sum(term_sum, axis=1)
    npos_row = jnp.sum(npos, axis=1)
    acc = jnp.sum(row_sum / npos_row)

    contrib = jnp.reshape(acc * (1.0 / _ROWS), (1, 1))
    @pl.when(i == 0)
    def _():
        o_ref[...] = contrib
    @pl.when(i > 0)
    def _():
        o_ref[...] = o_ref[...] + contrib


def kernel(preds, labels):
    nb = _ROWS // _BR
    tile_specs = [
        pl.BlockSpec((_BR, 128), (lambda i, g=g: (i, g))) for g in range(8)
    ]
    out = pl.pallas_call(
        _body,
        grid=(nb,),
        in_specs=tile_specs + tile_specs,
        out_specs=pl.BlockSpec((1, 1), lambda i: (0, 0)),
        out_shape=jax.ShapeDtypeStruct((1, 1), jnp.float32),
    )(*([preds] * 8 + [labels] * 8))
    return out[0, 0]


# speedup vs baseline: 6.6847x; 1.1758x over previous
import jax
import jax.numpy as jnp
from jax.experimental import pallas as pl

_N = 1024
_C = 1000
_ROWS = 16384
_BR = 128


def _body(*refs):
    o_ref = refs[16]
    i = pl.program_id(0)
    lane = jax.lax.broadcasted_iota(jnp.int32, (_BR, 128), 1)

    pad7 = lane >= (_C - 7 * 128)
    K = [refs[g][...] for g in range(8)]
    K[7] = jnp.where(pad7, -jnp.inf, K[7])
    L = [refs[8 + g][...] for g in range(8)]
    L[7] = jnp.where(pad7, 0.0, L[7])
    V = [lane * 2 + (g * 256) + L[g].astype(jnp.int32)
         for g in range(8)]

    for ksz_log in range(1, 11):
        ksz = 1 << ksz_log
        for j_log in range(ksz_log - 1, -1, -1):
            j = 1 << j_log
            if j < 8:
                for g in range(8):
                    if g & j:
                        continue
                    h = g ^ j
                    a, b, va, vb = K[g], K[h], V[g], V[h]
                    after = a < b
                    if ksz < 8:
                        swap = (~after) if (g & ksz) else after
                    else:
                        asc = (lane & (ksz // 8)) == 0
                        swap = after == asc
                    K[g] = jnp.where(swap, b, a)
                    K[h] = jnp.where(swap, a, b)
                    V[g] = jnp.where(swap, vb, va)
                    V[h] = jnp.where(swap, va, vb)
            else:
                jl = j // 8
                asc = (lane & (ksz // 8)) == 0
                low = (lane & jl) == 0
                for g in range(8):
                    k, v = K[g], V[g]
                    pk = jnp.where(low, jnp.roll(k, -jl, axis=1),
                                   jnp.roll(k, jl, axis=1))
                    pv = jnp.where(low, jnp.roll(v, -jl, axis=1),
                                   jnp.roll(v, jl, axis=1))
                    after = k < pk
                    take = after == (asc == low)
                    K[g] = jnp.where(take, pk, k)
                    V[g] = jnp.where(take, pv, v)

    ls = [(V[g] & 1).astype(jnp.float32) for g in range(8)]
    run = ls[0]
    cums = [run]
    for g in range(1, 8):
        run = run + ls[g]
        cums.append(run)
    inc = run
    for d in (1, 2, 4, 8, 16, 32, 64):
        inc = inc + jnp.where(lane >= d, jnp.roll(inc, d, axis=1), 0.0)
    excl = inc - run

    term_sum = jnp.zeros_like(run)
    npos = run * 0.0
    for g in range(8):
        pos = (lane * 8 + g + 1).astype(jnp.float32)
        cum = cums[g] + excl
        term_sum = term_sum + ls[g] * (cum / pos)
        npos = npos + ls[g]
    row_sum = jnp.sum(term_sum, axis=1)
    npos_row = jnp.sum(npos, axis=1)
    acc = jnp.sum(row_sum / npos_row)

    contrib = jnp.reshape(acc * (1.0 / _ROWS), (1, 1))
    @pl.when(i == 0)
    def _():
        o_ref[...] = contrib
    @pl.when(i > 0)
    def _():
        o_ref[...] = o_ref[...] + contrib


def kernel(preds, labels):
    nb = _ROWS // _BR
    tile_specs = [
        pl.BlockSpec((_BR, 128), (lambda i, g=g: (i, g))) for g in range(8)
    ]
    out = pl.pallas_call(
        _body,
        grid=(nb,),
        in_specs=tile_specs + tile_specs,
        out_specs=pl.BlockSpec((1, 1), lambda i: (0, 0)),
        out_shape=jax.ShapeDtypeStruct((1, 1), jnp.float32),
    )(*([preds] * 8 + [labels] * 8))
    return out[0, 0]
